# Initial kernel scaffold; baseline (speedup 1.0000x reference)
import jax
import jax.numpy as jnp
from jax.experimental import pallas as pl


def kernel(X1, L1_indices, L1_values, batch, W1, W2, W3, W4):
    raise NotImplementedError("write your pallas kernel here")


# speedup vs baseline: 11.9375x; 11.9375x over previous
import functools

import jax
import jax.numpy as jnp
from jax import lax
from jax.experimental import pallas as pl
from jax.experimental.pallas import tpu as pltpu
from jax.experimental.pallas import tpu_sc as plsc

_N = 10000
_NNZ = 320000
_NUM_GRAPHS = 64
_NEG = 0.01

_NT = 32
_CH = 128
_NCHUNK = 79
_PER_TILE = _CH * _NCHUNK
_PAD_NNZ = _PER_TILE * _NT
_RPS = 624
_REM0 = 16 * _RPS
_REM = _N - _REM0


def _make_spmm(width):
  mesh = plsc.VectorSubcoreMesh(core_axis_name="c", subcore_axis_name="s")

  @functools.partial(
      pl.kernel,
      out_type=jax.ShapeDtypeStruct((2, _N, width), jnp.float32),
      mesh=mesh,
      scratch_types=[
          pltpu.VMEM((_NCHUNK, _CH), jnp.int32),
          pltpu.VMEM((_NCHUNK, _CH), jnp.int32),
          pltpu.VMEM((_PER_TILE,), jnp.float32),
          pltpu.VMEM((_CH, width), jnp.float32),
          pltpu.VMEM_SHARED((_N, width), jnp.float32),
          pltpu.SemaphoreType.DMA,
      ],
      compiler_params=pltpu.CompilerParams(use_tc_tiling_on_sc=False),
      name=f"spmm_sc_w{width}",
  )
  def spmm(y, colr, rowr, valr, zeros, out, colv, rowv, valv, gbuf, acc, sem):
    c = lax.axis_index("c")
    s = lax.axis_index("s")
    wid = s * 2 + c
    r0 = s * _RPS
    pltpu.sync_copy(zeros.at[pl.ds(r0, _RPS)], acc.at[pl.ds(r0, _RPS)])

    @pl.when(s == 15)
    def _():
      pltpu.sync_copy(zeros.at[pl.ds(_REM0, _REM)], acc.at[pl.ds(_REM0, _REM)])
    pltpu.sync_copy(colr.at[wid], colv)
    pltpu.sync_copy(rowr.at[wid], rowv)
    pltpu.sync_copy(valr.at[wid], valv)
    plsc.subcore_barrier()

    def chunk(j, carry):
      pltpu.async_copy(y.at[colv.at[j]], gbuf, sem).wait()

      def scale(t, carry2):
        vv = valv[pl.ds(j * _CH + t * 16, 16)]
        for e in range(16):
          v = vv[e]
          r = t * 16 + e
          for k in range(width // 16):
            sl = pl.ds(k * 16, 16)
            gbuf[r, sl] = gbuf[r, sl] * v
        return carry2

      lax.fori_loop(0, _CH // 16, scale, 0)
      pltpu.sync_copy(gbuf, acc.at[rowv.at[j]], add=True)
      return carry

    lax.fori_loop(0, _NCHUNK, chunk, 0)
    plsc.subcore_barrier()
    pltpu.sync_copy(acc.at[pl.ds(r0, _RPS)], out.at[c, pl.ds(r0, _RPS)])

    @pl.when(s == 15)
    def _():
      pltpu.sync_copy(acc.at[pl.ds(_REM0, _REM)],
                      out.at[c, pl.ds(_REM0, _REM)])

  return spmm


_spmm32 = _make_spmm(32)
_spmm16 = _make_spmm(16)

_BLK = 2000


def _mm_first(x, w):
  def body(x_ref, w_ref, o_ref):
    o_ref[...] = jnp.dot(x_ref[...], w_ref[...],
                         preferred_element_type=jnp.float32)

  return pl.pallas_call(
      body,
      grid=(_N // _BLK,),
      in_specs=[
          pl.BlockSpec((_BLK, 128), lambda i: (i, 0)),
          pl.BlockSpec((128, 32), lambda i: (0, 0)),
      ],
      out_specs=pl.BlockSpec((_BLK, 32), lambda i: (i, 0)),
      out_shape=jax.ShapeDtypeStruct((_N, 32), jnp.float32),
      name="mm_first",
  )(x, w)


def _relu_mm(p, w, wo):
  def body(p_ref, w_ref, o_ref):
    h = p_ref[0] + p_ref[1]
    h = jnp.where(h >= 0, h, h * _NEG)
    o_ref[...] = jnp.dot(h, w_ref[...], preferred_element_type=jnp.float32)

  return pl.pallas_call(
      body,
      grid=(_N // _BLK,),
      in_specs=[
          pl.BlockSpec((2, _BLK, 32), lambda i: (0, i, 0)),
          pl.BlockSpec((32, wo), lambda i: (0, 0)),
      ],
      out_specs=pl.BlockSpec((_BLK, wo), lambda i: (i, 0)),
      out_shape=jax.ShapeDtypeStruct((_N, wo), jnp.float32),
      name="relu_mm",
  )(p, w)


def _pool_softmax(p, batch2d):
  def body(p_ref, b_ref, o_ref):
    h = p_ref[0] + p_ref[1]
    h = jnp.where(h >= 0, h, h * _NEG)
    ccol = lax.broadcasted_iota(jnp.int32, (_N, 16), 1)
    h = jnp.where(ccol == 10, 1.0, h)
    gids = lax.broadcasted_iota(jnp.int32, (_NUM_GRAPHS, _N), 0)
    onehot = (gids == b_ref[...]).astype(jnp.float32)
    pooled = jnp.dot(onehot, h, preferred_element_type=jnp.float32)
    counts = jnp.maximum(pooled[:, 10:11], 1.0)
    means = pooled / counts
    gcol = lax.broadcasted_iota(jnp.int32, (_NUM_GRAPHS, 16), 1)
    valid = gcol < 10
    z = jnp.where(valid, means, -1e30)
    z = z - jnp.max(z, axis=1, keepdims=True)
    ez = jnp.where(valid, jnp.exp(z), 0.0)
    o_ref[...] = ez / jnp.sum(ez, axis=1, keepdims=True)

  return pl.pallas_call(
      body,
      in_specs=[
          pl.BlockSpec((2, _N, 16), lambda: (0, 0, 0)),
          pl.BlockSpec((1, _N), lambda: (0, 0)),
      ],
      out_specs=pl.BlockSpec((_NUM_GRAPHS, 16), lambda: (0, 0)),
      out_shape=jax.ShapeDtypeStruct((_NUM_GRAPHS, 16), jnp.float32),
      name="pool_softmax",
  )(p, batch2d)


def kernel(X1, L1_indices, L1_values, batch, W1, W2, W3, W4):
  row = L1_indices[0]
  col = L1_indices[1]
  npad = _PAD_NNZ - _NNZ
  pad_idx = (jnp.arange(npad, dtype=jnp.int32) * 131) % _N
  colp = jnp.concatenate([col, pad_idx]).reshape(_NT, _NCHUNK, _CH)
  rowp = jnp.concatenate([row, pad_idx]).reshape(_NT, _NCHUNK, _CH)
  valp = jnp.concatenate(
      [L1_values, jnp.zeros((npad,), jnp.float32)]).reshape(_NT, _PER_TILE)
  zeros32 = jnp.zeros((_N, 32), jnp.float32)
  zeros16 = jnp.zeros((_N, 16), jnp.float32)
  w4p = jnp.zeros((32, 16), jnp.float32).at[:, :10].set(W4)

  y = _mm_first(X1, W1)
  p = _spmm32(y, colp, rowp, valp, zeros32)
  y = _relu_mm(p, W2, 32)
  p = _spmm32(y, colp, rowp, valp, zeros32)
  y = _relu_mm(p, W3, 32)
  p = _spmm32(y, colp, rowp, valp, zeros32)
  y = _relu_mm(p, w4p, 16)
  p = _spmm16(y, colp, rowp, valp, zeros16)
  out = _pool_softmax(p, batch.reshape(1, _N))
  return out[:, :10]
